# baseline (speedup 1.0000x reference)
import functools

import jax
import jax.numpy as jnp
from jax import lax
from jax.experimental import pallas as pl
from jax.experimental.pallas import tpu as pltpu
from jax.experimental.pallas import tpu_sc as plsc

D = 128
N_NODES = 10000
NC, NS = 2, 16
NW = NC * NS
ROWS_PER_TILE = 632
N_PAD = NS * ROWS_PER_TILE
CHUNK = 128
N_EDGES = 320000
CPT = 80
GB = 128
GPT = CPT * CHUNK // GB
HALF = 16
GHALF = HALF * CHUNK // GB
EDGES_PAD = NW * CPT * CHUNK

ROW_BLK = 1000


def _mm_body(h_ref, w_ref, o_ref):
    o_ref[...] = lax.dot_general(
        h_ref[...], w_ref[...], (((1,), (1,)), ((), ())),
        preferred_element_type=jnp.float32)


def _fin_body(h_ref, a0_ref, a1_ref, g_ref, b_ref, o_ref):
    agg = a0_ref[...] + a1_ref[...]
    ge = 0.5 * agg * (1.0 + lax.erf(agg * 0.7071067811865476))
    x = h_ref[...] + ge
    mu = jnp.mean(x, axis=1, keepdims=True)
    xc = x - mu
    var = jnp.mean(xc * xc, axis=1, keepdims=True)
    y = xc * lax.rsqrt(var + 1e-5)
    o_ref[...] = y * g_ref[...] + b_ref[...]


def _sc_body(m_hbm, src_hbm, dst_hbm, zero_hbm, out_hbm,
             src_v, dst_v, rows_v, sem, shared):
    cid = lax.axis_index("c")
    sid = lax.axis_index("s")
    wid = sid * NC + cid
    row0 = sid * ROWS_PER_TILE

    pltpu.sync_copy(zero_hbm.at[pl.ds(row0, ROWS_PER_TILE)],
                    shared.at[pl.ds(row0, ROWS_PER_TILE)])
    plsc.subcore_barrier()

    for h in range(CPT // HALF):
        pltpu.sync_copy(src_hbm.at[wid, pl.ds(h * GHALF, GHALF)], src_v)
        pltpu.sync_copy(dst_hbm.at[wid, pl.ds(h * HALF, HALF)], dst_v)

        def step(k, carry):
            pltpu.async_copy(m_hbm.at[src_v.at[k]], rows_v, sem).wait()
            for b in range(GB // CHUNK):
                pltpu.sync_copy(
                    rows_v.at[pl.ds(b * CHUNK, CHUNK)],
                    shared.at[dst_v.at[k * (GB // CHUNK) + b]],
                    add=True)
            return carry

        lax.fori_loop(0, GHALF, step, 0, unroll=False)
    plsc.subcore_barrier()
    pltpu.sync_copy(shared.at[pl.ds(row0, ROWS_PER_TILE)],
                    out_hbm.at[cid, pl.ds(row0, ROWS_PER_TILE)])


_sc_scatter = pl.kernel(
    _sc_body,
    out_type=jax.ShapeDtypeStruct((NC, N_PAD, D), jnp.float32),
    mesh=plsc.VectorSubcoreMesh(core_axis_name="c", subcore_axis_name="s"),
    scratch_types=[
        pltpu.VMEM((GHALF, GB), jnp.int32),
        pltpu.VMEM((HALF, CHUNK), jnp.int32),
        pltpu.VMEM((GB, D), jnp.float32),
        pltpu.SemaphoreType.DMA,
        pltpu.VMEM_SHARED((N_PAD, D), jnp.float32),
    ],
)


def kernel(H, src, dst, W, gamma, beta):
    H2 = H.reshape(N_NODES, D)

    m = pl.pallas_call(
        _mm_body,
        out_shape=jax.ShapeDtypeStruct((N_NODES, D), jnp.float32),
        grid=(N_NODES // ROW_BLK,),
        in_specs=[pl.BlockSpec((ROW_BLK, D), lambda i: (i, 0)),
                  pl.BlockSpec((D, D), lambda i: (0, 0))],
        out_specs=pl.BlockSpec((ROW_BLK, D), lambda i: (i, 0)),
    )(H2, W)

    pad = EDGES_PAD - src.shape[0]
    src3 = jnp.concatenate(
        [src.astype(jnp.int32), jnp.zeros((pad,), jnp.int32)]
    ).reshape(NW, GPT, GB)
    dst3 = jnp.concatenate(
        [dst.astype(jnp.int32), jnp.full((pad,), N_NODES, jnp.int32)]
    ).reshape(NW, CPT, CHUNK)
    zeros = jnp.zeros((N_PAD, D), jnp.float32)

    parts = _sc_scatter(m, src3, dst3, zeros)

    out = pl.pallas_call(
        _fin_body,
        out_shape=jax.ShapeDtypeStruct((N_NODES, D), jnp.float32),
        grid=(N_NODES // ROW_BLK,),
        in_specs=[pl.BlockSpec((ROW_BLK, D), lambda i: (i, 0)),
                  pl.BlockSpec((ROW_BLK, D), lambda i: (i, 0)),
                  pl.BlockSpec((ROW_BLK, D), lambda i: (i, 0)),
                  pl.BlockSpec((1, D), lambda i: (0, 0)),
                  pl.BlockSpec((1, D), lambda i: (0, 0))],
        out_specs=pl.BlockSpec((ROW_BLK, D), lambda i: (i, 0)),
    )(H2, parts[0, :N_NODES], parts[1, :N_NODES],
      gamma.reshape(1, D), beta.reshape(1, D))

    return out.reshape(1, N_NODES, D)


# speedup vs baseline: 1.0263x; 1.0263x over previous
import functools

import jax
import jax.numpy as jnp
from jax import lax
from jax.experimental import pallas as pl
from jax.experimental.pallas import tpu as pltpu
from jax.experimental.pallas import tpu_sc as plsc

D = 128
N_NODES = 10000
NC, NS = 2, 16
NW = NC * NS
ROWS_PER_TILE = 632
N_PAD = NS * ROWS_PER_TILE
CHUNK = 128
N_EDGES = 320000
CPT = 80
GB = 128
GPT = CPT * CHUNK // GB
HALF = 80
GHALF = HALF * CHUNK // GB
EDGES_PAD = NW * CPT * CHUNK

ROW_BLK = 1000


def _mm_body(h_ref, w_ref, o_ref):
    o_ref[...] = lax.dot_general(
        h_ref[...], w_ref[...], (((1,), (1,)), ((), ())),
        preferred_element_type=jnp.float32)


def _fin_body(h_ref, a0_ref, a1_ref, g_ref, b_ref, o_ref):
    agg = a0_ref[...] + a1_ref[...]
    ge = 0.5 * agg * (1.0 + lax.erf(agg * 0.7071067811865476))
    x = h_ref[...] + ge
    mu = jnp.mean(x, axis=1, keepdims=True)
    xc = x - mu
    var = jnp.mean(xc * xc, axis=1, keepdims=True)
    y = xc * lax.rsqrt(var + 1e-5)
    o_ref[...] = y * g_ref[...] + b_ref[...]


def _sc_body(m_hbm, src_hbm, dst_hbm, zero_hbm, out_hbm,
             src_v, dst_v, rows_v, sem, shared):
    cid = lax.axis_index("c")
    sid = lax.axis_index("s")
    wid = sid * NC + cid
    row0 = sid * ROWS_PER_TILE

    pltpu.sync_copy(zero_hbm.at[pl.ds(row0, ROWS_PER_TILE)],
                    shared.at[pl.ds(row0, ROWS_PER_TILE)])
    plsc.subcore_barrier()

    for h in range(CPT // HALF):
        pltpu.sync_copy(src_hbm.at[wid, pl.ds(h * GHALF, GHALF)], src_v)
        pltpu.sync_copy(dst_hbm.at[wid, pl.ds(h * HALF, HALF)], dst_v)

        def step(k, carry):
            pltpu.async_copy(m_hbm.at[src_v.at[k]], rows_v, sem).wait()
            for b in range(GB // CHUNK):
                pltpu.sync_copy(
                    rows_v.at[pl.ds(b * CHUNK, CHUNK)],
                    shared.at[dst_v.at[k * (GB // CHUNK) + b]],
                    add=True)
            return carry

        lax.fori_loop(0, GHALF, step, 0, unroll=False)
    plsc.subcore_barrier()
    pltpu.sync_copy(shared.at[pl.ds(row0, ROWS_PER_TILE)],
                    out_hbm.at[cid, pl.ds(row0, ROWS_PER_TILE)])


_sc_scatter = pl.kernel(
    _sc_body,
    out_type=jax.ShapeDtypeStruct((NC, N_PAD, D), jnp.float32),
    mesh=plsc.VectorSubcoreMesh(core_axis_name="c", subcore_axis_name="s"),
    scratch_types=[
        pltpu.VMEM((GHALF, GB), jnp.int32),
        pltpu.VMEM((HALF, CHUNK), jnp.int32),
        pltpu.VMEM((GB, D), jnp.float32),
        pltpu.SemaphoreType.DMA,
        pltpu.VMEM_SHARED((N_PAD, D), jnp.float32),
    ],
)


def kernel(H, src, dst, W, gamma, beta):
    H2 = H.reshape(N_NODES, D)

    m = pl.pallas_call(
        _mm_body,
        out_shape=jax.ShapeDtypeStruct((N_NODES, D), jnp.float32),
        grid=(N_NODES // ROW_BLK,),
        in_specs=[pl.BlockSpec((ROW_BLK, D), lambda i: (i, 0)),
                  pl.BlockSpec((D, D), lambda i: (0, 0))],
        out_specs=pl.BlockSpec((ROW_BLK, D), lambda i: (i, 0)),
    )(H2, W)

    pad = EDGES_PAD - src.shape[0]
    src3 = jnp.concatenate(
        [src.astype(jnp.int32), jnp.zeros((pad,), jnp.int32)]
    ).reshape(NW, GPT, GB)
    dst3 = jnp.concatenate(
        [dst.astype(jnp.int32), jnp.full((pad,), N_NODES, jnp.int32)]
    ).reshape(NW, CPT, CHUNK)
    zeros = jnp.zeros((N_PAD, D), jnp.float32)

    parts = _sc_scatter(m, src3, dst3, zeros)

    out = pl.pallas_call(
        _fin_body,
        out_shape=jax.ShapeDtypeStruct((N_NODES, D), jnp.float32),
        grid=(N_NODES // ROW_BLK,),
        in_specs=[pl.BlockSpec((ROW_BLK, D), lambda i: (i, 0)),
                  pl.BlockSpec((ROW_BLK, D), lambda i: (i, 0)),
                  pl.BlockSpec((ROW_BLK, D), lambda i: (i, 0)),
                  pl.BlockSpec((1, D), lambda i: (0, 0)),
                  pl.BlockSpec((1, D), lambda i: (0, 0))],
        out_specs=pl.BlockSpec((ROW_BLK, D), lambda i: (i, 0)),
    )(H2, parts[0, :N_NODES], parts[1, :N_NODES],
      gamma.reshape(1, D), beta.reshape(1, D))

    return out.reshape(1, N_NODES, D)


# speedup vs baseline: 1.0269x; 1.0006x over previous
import functools

import jax
import jax.numpy as jnp
from jax import lax
from jax.experimental import pallas as pl
from jax.experimental.pallas import tpu as pltpu
from jax.experimental.pallas import tpu_sc as plsc

D = 128
N_NODES = 10000
NC, NS = 2, 16
NW = NC * NS
ROWS_PER_TILE = 632
N_PAD = NS * ROWS_PER_TILE
CHUNK = 128
N_EDGES = 320000
CPT = 80
GB = 128
GPT = CPT * CHUNK // GB
HALF = 80
GHALF = HALF * CHUNK // GB
EDGES_PAD = NW * CPT * CHUNK

ROW_BLK = 1000


def _mm_body(h_ref, w_ref, o_ref):
    o_ref[...] = lax.dot_general(
        h_ref[...], w_ref[...], (((1,), (1,)), ((), ())),
        preferred_element_type=jnp.float32)


def _fin_body(h_ref, a0_ref, a1_ref, g_ref, b_ref, o_ref):
    agg = a0_ref[...] + a1_ref[...]
    ge = 0.5 * agg * (1.0 + lax.erf(agg * 0.7071067811865476))
    x = h_ref[...] + ge
    mu = jnp.mean(x, axis=1, keepdims=True)
    xc = x - mu
    var = jnp.mean(xc * xc, axis=1, keepdims=True)
    y = xc * lax.rsqrt(var + 1e-5)
    o_ref[...] = y * g_ref[...] + b_ref[...]


def _sc_body(m_hbm, src_hbm, dst_hbm, zero_hbm, out_hbm,
             src_v, dst_v, rows_v, sem, shared):
    cid = lax.axis_index("c")
    sid = lax.axis_index("s")
    wid = sid * NC + cid
    row0 = sid * ROWS_PER_TILE

    pltpu.sync_copy(zero_hbm.at[pl.ds(row0, ROWS_PER_TILE)],
                    shared.at[pl.ds(row0, ROWS_PER_TILE)])
    plsc.subcore_barrier()

    for h in range(CPT // HALF):
        pltpu.sync_copy(src_hbm.at[wid, pl.ds(h * GHALF, GHALF)], src_v)
        pltpu.sync_copy(dst_hbm.at[wid, pl.ds(h * HALF, HALF)], dst_v)

        def step(k, carry):
            pltpu.async_copy(m_hbm.at[src_v.at[k]], rows_v, sem).wait()
            for b in range(GB // CHUNK):
                src_rows = (rows_v if GB == CHUNK
                            else rows_v.at[pl.ds(b * CHUNK, CHUNK)])
                pltpu.sync_copy(
                    src_rows,
                    shared.at[dst_v.at[k * (GB // CHUNK) + b]],
                    add=True)
            return carry

        lax.fori_loop(0, GHALF, step, 0, unroll=False)
    plsc.subcore_barrier()
    pltpu.sync_copy(shared.at[pl.ds(row0, ROWS_PER_TILE)],
                    out_hbm.at[cid, pl.ds(row0, ROWS_PER_TILE)])


_sc_scatter = pl.kernel(
    _sc_body,
    out_type=jax.ShapeDtypeStruct((NC, N_PAD, D), jnp.float32),
    mesh=plsc.VectorSubcoreMesh(core_axis_name="c", subcore_axis_name="s"),
    scratch_types=[
        pltpu.VMEM((GHALF, GB), jnp.int32),
        pltpu.VMEM((HALF, CHUNK), jnp.int32),
        pltpu.VMEM((GB, D), jnp.float32),
        pltpu.SemaphoreType.DMA,
        pltpu.VMEM_SHARED((N_PAD, D), jnp.float32),
    ],
)


def kernel(H, src, dst, W, gamma, beta):
    H2 = H.reshape(N_NODES, D)

    m = pl.pallas_call(
        _mm_body,
        out_shape=jax.ShapeDtypeStruct((N_NODES, D), jnp.float32),
        grid=(N_NODES // ROW_BLK,),
        in_specs=[pl.BlockSpec((ROW_BLK, D), lambda i: (i, 0)),
                  pl.BlockSpec((D, D), lambda i: (0, 0))],
        out_specs=pl.BlockSpec((ROW_BLK, D), lambda i: (i, 0)),
    )(H2, W)

    pad = EDGES_PAD - src.shape[0]
    src3 = jnp.concatenate(
        [src.astype(jnp.int32), jnp.zeros((pad,), jnp.int32)]
    ).reshape(NW, GPT, GB)
    dst3 = jnp.concatenate(
        [dst.astype(jnp.int32), jnp.full((pad,), N_NODES, jnp.int32)]
    ).reshape(NW, CPT, CHUNK)
    zeros = jnp.zeros((N_PAD, D), jnp.float32)

    parts = _sc_scatter(m, src3, dst3, zeros)

    out = pl.pallas_call(
        _fin_body,
        out_shape=jax.ShapeDtypeStruct((N_NODES, D), jnp.float32),
        grid=(N_NODES // ROW_BLK,),
        in_specs=[pl.BlockSpec((ROW_BLK, D), lambda i: (i, 0)),
                  pl.BlockSpec((ROW_BLK, D), lambda i: (i, 0)),
                  pl.BlockSpec((ROW_BLK, D), lambda i: (i, 0)),
                  pl.BlockSpec((1, D), lambda i: (0, 0)),
                  pl.BlockSpec((1, D), lambda i: (0, 0))],
        out_specs=pl.BlockSpec((ROW_BLK, D), lambda i: (i, 0)),
    )(H2, parts[0, :N_NODES], parts[1, :N_NODES],
      gamma.reshape(1, D), beta.reshape(1, D))

    return out.reshape(1, N_NODES, D)


# speedup vs baseline: 2.5513x; 2.4844x over previous
import functools

import jax
import jax.numpy as jnp
from jax import lax
from jax.experimental import pallas as pl
from jax.experimental.pallas import tpu as pltpu
from jax.experimental.pallas import tpu_sc as plsc

D = 128
N_NODES = 10000
NC, NS = 2, 16
NW = NC * NS
ROWS_PER_TILE = 632
N_PAD = NS * ROWS_PER_TILE
CHUNK = 128
N_EDGES = 320000
CPT = 80
GB = 128
GPT = CPT * CHUNK // GB
HALF = 80
GHALF = HALF * CHUNK // GB
EDGES_PAD = NW * CPT * CHUNK

ROW_BLK = 1000


def _mm_body(h_ref, w_ref, o_ref):
    o_ref[...] = lax.dot_general(
        h_ref[...], w_ref[...], (((1,), (1,)), ((), ())),
        preferred_element_type=jnp.float32)


def _fin_body(h_ref, a0_ref, a1_ref, g_ref, b_ref, o_ref):
    agg = a0_ref[...] + a1_ref[...]
    ge = 0.5 * agg * (1.0 + lax.erf(agg * 0.7071067811865476))
    x = h_ref[...] + ge
    mu = jnp.mean(x, axis=1, keepdims=True)
    xc = x - mu
    var = jnp.mean(xc * xc, axis=1, keepdims=True)
    y = xc * lax.rsqrt(var + 1e-5)
    o_ref[...] = y * g_ref[...] + b_ref[...]


def _sc_body(m_hbm, src_hbm, dst_hbm, zero_hbm, out_hbm,
             src_v, dst_v, rows_v, sem, shared):
    cid = lax.axis_index("c")
    sid = lax.axis_index("s")
    wid = sid * NC + cid
    row0 = sid * ROWS_PER_TILE

    pltpu.sync_copy(zero_hbm.at[pl.ds(row0, ROWS_PER_TILE)],
                    shared.at[pl.ds(row0, ROWS_PER_TILE)])
    plsc.subcore_barrier()

    for h in range(CPT // HALF):
        pltpu.sync_copy(src_hbm.at[wid, pl.ds(h * GHALF, GHALF)], src_v)
        pltpu.sync_copy(dst_hbm.at[wid, pl.ds(h * HALF, HALF)], dst_v)

        def step(k, carry):
            pltpu.async_copy(m_hbm.at[src_v.at[k]], rows_v, sem).wait()
            for b in range(GB // CHUNK):
                src_rows = (rows_v if GB == CHUNK
                            else rows_v.at[pl.ds(b * CHUNK, CHUNK)])
                pltpu.sync_copy(
                    src_rows,
                    shared.at[dst_v.at[k * (GB // CHUNK) + b]],
                    add=True)
            return carry

        lax.fori_loop(0, GHALF, step, 0, unroll=False)
    plsc.subcore_barrier()
    pltpu.sync_copy(shared.at[pl.ds(row0, ROWS_PER_TILE)],
                    out_hbm.at[cid, pl.ds(row0, ROWS_PER_TILE)])


_sc_scatter = pl.kernel(
    _sc_body,
    out_type=jax.ShapeDtypeStruct((NC, N_PAD, D), jnp.float32),
    mesh=plsc.VectorSubcoreMesh(core_axis_name="c", subcore_axis_name="s"),
    scratch_types=[
        pltpu.VMEM((GHALF, GB), jnp.int32),
        pltpu.VMEM((HALF, CHUNK), jnp.int32),
        pltpu.VMEM((GB, D), jnp.float32),
        pltpu.SemaphoreType.DMA,
        pltpu.VMEM_SHARED((N_PAD, D), jnp.float32),
    ],
)


def kernel(H, src, dst, W, gamma, beta):
    H2 = H.reshape(N_NODES, D)

    m = pl.pallas_call(
        _mm_body,
        out_shape=jax.ShapeDtypeStruct((N_NODES, D), jnp.float32),
        grid=(N_NODES // ROW_BLK,),
        in_specs=[pl.BlockSpec((ROW_BLK, D), lambda i: (i, 0)),
                  pl.BlockSpec((D, D), lambda i: (0, 0))],
        out_specs=pl.BlockSpec((ROW_BLK, D), lambda i: (i, 0)),
    )(H2, W)

    pad = EDGES_PAD - src.shape[0]
    pad_iota = lax.iota(jnp.int32, pad)
    src3 = jnp.concatenate(
        [src.astype(jnp.int32), pad_iota % N_NODES]
    ).reshape(NW, GPT, GB)
    dst3 = jnp.concatenate(
        [dst.astype(jnp.int32), N_NODES + pad_iota % (N_PAD - N_NODES)]
    ).reshape(NW, CPT, CHUNK)
    zeros = jnp.zeros((N_PAD, D), jnp.float32)

    parts = _sc_scatter(m, src3, dst3, zeros)

    out = pl.pallas_call(
        _fin_body,
        out_shape=jax.ShapeDtypeStruct((N_NODES, D), jnp.float32),
        grid=(N_NODES // ROW_BLK,),
        in_specs=[pl.BlockSpec((ROW_BLK, D), lambda i: (i, 0)),
                  pl.BlockSpec((ROW_BLK, D), lambda i: (i, 0)),
                  pl.BlockSpec((ROW_BLK, D), lambda i: (i, 0)),
                  pl.BlockSpec((1, D), lambda i: (0, 0)),
                  pl.BlockSpec((1, D), lambda i: (0, 0))],
        out_specs=pl.BlockSpec((ROW_BLK, D), lambda i: (i, 0)),
    )(H2, parts[0, :N_NODES], parts[1, :N_NODES],
      gamma.reshape(1, D), beta.reshape(1, D))

    return out.reshape(1, N_NODES, D)


# speedup vs baseline: 3.1885x; 1.2498x over previous
import functools

import jax
import jax.numpy as jnp
from jax import lax
from jax.experimental import pallas as pl
from jax.experimental.pallas import tpu as pltpu
from jax.experimental.pallas import tpu_sc as plsc

D = 128
N_NODES = 10000
NC, NS = 2, 16
NW = NC * NS
ROWS_PER_TILE = 632
N_PAD = NS * ROWS_PER_TILE
CHUNK = 128
N_EDGES = 320000
CPT = 80
GB = 128
GPT = CPT * CHUNK // GB
HALF = 40
GHALF = HALF * CHUNK // GB
EDGES_PAD = NW * CPT * CHUNK

ROW_BLK = 1000


def _mm_body(h_ref, w_ref, o_ref):
    o_ref[...] = lax.dot_general(
        h_ref[...], w_ref[...], (((1,), (1,)), ((), ())),
        preferred_element_type=jnp.float32)


def _fin_body(h_ref, a0_ref, a1_ref, g_ref, b_ref, o_ref):
    agg = a0_ref[...] + a1_ref[...]
    ge = 0.5 * agg * (1.0 + lax.erf(agg * 0.7071067811865476))
    x = h_ref[...] + ge
    mu = jnp.mean(x, axis=1, keepdims=True)
    xc = x - mu
    var = jnp.mean(xc * xc, axis=1, keepdims=True)
    y = xc * lax.rsqrt(var + 1e-5)
    o_ref[...] = y * g_ref[...] + b_ref[...]


def _sc_body(m_hbm, src_hbm, dst_hbm, zero_hbm, out_hbm,
             src_v, dst_v, rows0_v, rows1_v, sem, sem_s, shared):
    cid = lax.axis_index("c")
    sid = lax.axis_index("s")
    wid = sid * NC + cid
    row0 = sid * ROWS_PER_TILE

    pltpu.sync_copy(zero_hbm.at[pl.ds(row0, ROWS_PER_TILE)],
                    shared.at[pl.ds(row0, ROWS_PER_TILE)])
    plsc.subcore_barrier()

    rows = (rows0_v, rows1_v)

    def gath(e, b):
        pltpu.async_copy(m_hbm.at[src_v.at[e]], rows[b], sem)

    def wait_g(e, b):
        pltpu.make_async_copy(m_hbm.at[src_v.at[e]], rows[b], sem).wait()

    def scat(e, b):
        pltpu.async_copy(rows[b], shared.at[dst_v.at[e]], sem_s, add=True)

    def wait_s(e, b):
        pltpu.make_async_copy(rows[b], shared.at[dst_v.at[e]],
                              sem_s).wait()

    for h in range(CPT // HALF):
        pltpu.sync_copy(src_hbm.at[wid, pl.ds(h * HALF, HALF)], src_v)
        pltpu.sync_copy(dst_hbm.at[wid, pl.ds(h * HALF, HALF)], dst_v)

        gath(0, 0)
        wait_g(0, 0)
        scat(0, 0)
        gath(1, 1)

        def pair(i, carry):
            for b in range(2):
                e = 2 * i + 1 + b
                wait_g(e, 1 - b)
                scat(e, 1 - b)
                wait_s(e - 1, b)
                gath(e + 1, b)
            return carry

        lax.fori_loop(0, (HALF - 2) // 2, pair, 0, unroll=False)

        wait_g(HALF - 1, 1)
        scat(HALF - 1, 1)
        wait_s(HALF - 2, 0)
        wait_s(HALF - 1, 1)
    plsc.subcore_barrier()
    pltpu.sync_copy(shared.at[pl.ds(row0, ROWS_PER_TILE)],
                    out_hbm.at[cid, pl.ds(row0, ROWS_PER_TILE)])


_sc_scatter = pl.kernel(
    _sc_body,
    out_type=jax.ShapeDtypeStruct((NC, N_PAD, D), jnp.float32),
    mesh=plsc.VectorSubcoreMesh(core_axis_name="c", subcore_axis_name="s"),
    scratch_types=[
        pltpu.VMEM((HALF, CHUNK), jnp.int32),
        pltpu.VMEM((HALF, CHUNK), jnp.int32),
        pltpu.VMEM((CHUNK, D), jnp.float32),
        pltpu.VMEM((CHUNK, D), jnp.float32),
        pltpu.SemaphoreType.DMA,
        pltpu.SemaphoreType.DMA,
        pltpu.VMEM_SHARED((N_PAD, D), jnp.float32),
    ],
)


def kernel(H, src, dst, W, gamma, beta):
    H2 = H.reshape(N_NODES, D)

    m = pl.pallas_call(
        _mm_body,
        out_shape=jax.ShapeDtypeStruct((N_NODES, D), jnp.float32),
        grid=(N_NODES // ROW_BLK,),
        in_specs=[pl.BlockSpec((ROW_BLK, D), lambda i: (i, 0)),
                  pl.BlockSpec((D, D), lambda i: (0, 0))],
        out_specs=pl.BlockSpec((ROW_BLK, D), lambda i: (i, 0)),
    )(H2, W)

    pad = EDGES_PAD - src.shape[0]
    pad_iota = lax.iota(jnp.int32, pad)
    src3 = jnp.concatenate(
        [src.astype(jnp.int32), pad_iota % N_NODES]
    ).reshape(NW, CPT, CHUNK)
    dst3 = jnp.concatenate(
        [dst.astype(jnp.int32), N_NODES + pad_iota % (N_PAD - N_NODES)]
    ).reshape(NW, CPT, CHUNK)
    zeros = jnp.zeros((N_PAD, D), jnp.float32)

    parts = _sc_scatter(m, src3, dst3, zeros)

    out = pl.pallas_call(
        _fin_body,
        out_shape=jax.ShapeDtypeStruct((N_NODES, D), jnp.float32),
        grid=(N_NODES // ROW_BLK,),
        in_specs=[pl.BlockSpec((ROW_BLK, D), lambda i: (i, 0)),
                  pl.BlockSpec((ROW_BLK, D), lambda i: (i, 0)),
                  pl.BlockSpec((ROW_BLK, D), lambda i: (i, 0)),
                  pl.BlockSpec((1, D), lambda i: (0, 0)),
                  pl.BlockSpec((1, D), lambda i: (0, 0))],
        out_specs=pl.BlockSpec((ROW_BLK, D), lambda i: (i, 0)),
    )(H2, parts[0, :N_NODES], parts[1, :N_NODES],
      gamma.reshape(1, D), beta.reshape(1, D))

    return out.reshape(1, N_NODES, D)


# speedup vs baseline: 3.3136x; 1.0392x over previous
import functools

import jax
import jax.numpy as jnp
from jax import lax
from jax.experimental import pallas as pl
from jax.experimental.pallas import tpu as pltpu
from jax.experimental.pallas import tpu_sc as plsc

D = 128
N_NODES = 10000
NC, NS = 2, 16
NW = NC * NS
ROWS_PER_TILE = 632
N_PAD = NS * ROWS_PER_TILE
CHUNK = 128
N_EDGES = 320000
CPT = 80
GB = 128
GPT = CPT * CHUNK // GB
HALF = 40
GHALF = HALF * CHUNK // GB
EDGES_PAD = NW * CPT * CHUNK

ROW_BLK = 1000


def _mm_body(h_ref, w_ref, o_ref):
    o_ref[...] = lax.dot_general(
        h_ref[...], w_ref[...], (((1,), (1,)), ((), ())),
        preferred_element_type=jnp.float32)


def _fin_body(h_ref, a0_ref, a1_ref, g_ref, b_ref, o_ref):
    agg = a0_ref[0] + a1_ref[0]
    ge = 0.5 * agg * (1.0 + lax.erf(agg * 0.7071067811865476))
    x = h_ref[...] + ge
    mu = jnp.mean(x, axis=1, keepdims=True)
    xc = x - mu
    var = jnp.mean(xc * xc, axis=1, keepdims=True)
    y = xc * lax.rsqrt(var + 1e-5)
    o_ref[...] = y * g_ref[...] + b_ref[...]


def _sc_body(m_hbm, src_hbm, dst_hbm, zero_hbm, out_hbm,
             src_v, dst_v, rows0_v, rows1_v, sem, sem_s, shared):
    cid = lax.axis_index("c")
    sid = lax.axis_index("s")
    wid = sid * NC + cid
    row0 = sid * ROWS_PER_TILE

    pltpu.sync_copy(zero_hbm.at[pl.ds(row0, ROWS_PER_TILE)],
                    shared.at[pl.ds(row0, ROWS_PER_TILE)])
    plsc.subcore_barrier()

    rows = (rows0_v, rows1_v)

    def gath(e, b):
        pltpu.async_copy(m_hbm.at[src_v.at[e]], rows[b], sem)

    def wait_g(e, b):
        pltpu.make_async_copy(m_hbm.at[src_v.at[e]], rows[b], sem).wait()

    def scat(e, b):
        pltpu.async_copy(rows[b], shared.at[dst_v.at[e]], sem_s, add=True)

    def wait_s(e, b):
        pltpu.make_async_copy(rows[b], shared.at[dst_v.at[e]],
                              sem_s).wait()

    for h in range(CPT // HALF):
        pltpu.sync_copy(src_hbm.at[wid, pl.ds(h * HALF, HALF)], src_v)
        pltpu.sync_copy(dst_hbm.at[wid, pl.ds(h * HALF, HALF)], dst_v)

        gath(0, 0)
        wait_g(0, 0)
        scat(0, 0)
        gath(1, 1)

        def pair(i, carry):
            for b in range(2):
                e = 2 * i + 1 + b
                wait_g(e, 1 - b)
                scat(e, 1 - b)
                wait_s(e - 1, b)
                gath(e + 1, b)
            return carry

        lax.fori_loop(0, (HALF - 2) // 2, pair, 0, unroll=False)

        wait_g(HALF - 1, 1)
        scat(HALF - 1, 1)
        wait_s(HALF - 2, 0)
        wait_s(HALF - 1, 1)
    plsc.subcore_barrier()
    pltpu.sync_copy(shared.at[pl.ds(row0, ROWS_PER_TILE)],
                    out_hbm.at[cid, pl.ds(row0, ROWS_PER_TILE)])


_sc_scatter = pl.kernel(
    _sc_body,
    out_type=jax.ShapeDtypeStruct((NC, N_PAD, D), jnp.float32),
    mesh=plsc.VectorSubcoreMesh(core_axis_name="c", subcore_axis_name="s"),
    scratch_types=[
        pltpu.VMEM((HALF, CHUNK), jnp.int32),
        pltpu.VMEM((HALF, CHUNK), jnp.int32),
        pltpu.VMEM((CHUNK, D), jnp.float32),
        pltpu.VMEM((CHUNK, D), jnp.float32),
        pltpu.SemaphoreType.DMA,
        pltpu.SemaphoreType.DMA,
        pltpu.VMEM_SHARED((N_PAD, D), jnp.float32),
    ],
)


def kernel(H, src, dst, W, gamma, beta):
    H2 = H.reshape(N_NODES, D)

    m = pl.pallas_call(
        _mm_body,
        out_shape=jax.ShapeDtypeStruct((N_NODES, D), jnp.float32),
        grid=(N_NODES // ROW_BLK,),
        in_specs=[pl.BlockSpec((ROW_BLK, D), lambda i: (i, 0)),
                  pl.BlockSpec((D, D), lambda i: (0, 0))],
        out_specs=pl.BlockSpec((ROW_BLK, D), lambda i: (i, 0)),
    )(H2, W)

    pad = EDGES_PAD - src.shape[0]
    pad_iota = lax.iota(jnp.int32, pad)
    src3 = jnp.concatenate(
        [src.astype(jnp.int32), pad_iota % N_NODES]
    ).reshape(NW, CPT, CHUNK)
    dst3 = jnp.concatenate(
        [dst.astype(jnp.int32), N_NODES + pad_iota % (N_PAD - N_NODES)]
    ).reshape(NW, CPT, CHUNK)
    zeros = jnp.zeros((N_PAD, D), jnp.float32)

    parts = _sc_scatter(m, src3, dst3, zeros)

    out = pl.pallas_call(
        _fin_body,
        out_shape=jax.ShapeDtypeStruct((N_NODES, D), jnp.float32),
        grid=(N_NODES // ROW_BLK,),
        in_specs=[pl.BlockSpec((ROW_BLK, D), lambda i: (i, 0)),
                  pl.BlockSpec((1, ROW_BLK, D), lambda i: (0, i, 0)),
                  pl.BlockSpec((1, ROW_BLK, D), lambda i: (1, i, 0)),
                  pl.BlockSpec((1, D), lambda i: (0, 0)),
                  pl.BlockSpec((1, D), lambda i: (0, 0))],
        out_specs=pl.BlockSpec((ROW_BLK, D), lambda i: (i, 0)),
    )(H2, parts, parts, gamma.reshape(1, D), beta.reshape(1, D))

    return out.reshape(1, N_NODES, D)


# speedup vs baseline: 3.7980x; 1.1462x over previous
import functools

import jax
import jax.numpy as jnp
from jax import lax
from jax.experimental import pallas as pl
from jax.experimental.pallas import tpu as pltpu
from jax.experimental.pallas import tpu_sc as plsc

D = 128
N_NODES = 10000
NC, NS = 2, 16
NW = NC * NS
ROWS_PER_TILE = 632
N_PAD = NS * ROWS_PER_TILE
CHUNK = 128
N_EDGES = 320000
CPT = 80
GB = 128
GPT = CPT * CHUNK // GB
HALF = 40
GHALF = HALF * CHUNK // GB
EDGES_PAD = NW * CPT * CHUNK

ROW_BLK = 1000


def _mm_body(h_ref, w_ref, o_ref):
    o_ref[...] = lax.dot_general(
        h_ref[...], w_ref[...], (((1,), (1,)), ((), ())),
        preferred_element_type=jnp.float32)


def _fin_body(h_ref, a0_ref, a1_ref, g_ref, b_ref, o_ref):
    agg = a0_ref[0] + a1_ref[0]
    ge = 0.5 * agg * (1.0 + lax.erf(agg * 0.7071067811865476))
    x = h_ref[...] + ge
    mu = jnp.mean(x, axis=1, keepdims=True)
    xc = x - mu
    var = jnp.mean(xc * xc, axis=1, keepdims=True)
    y = xc * lax.rsqrt(var + 1e-5)
    o_ref[...] = y * g_ref[...] + b_ref[...]


def _sc_body(m_hbm, src_hbm, dst_hbm, zero_hbm, out_hbm,
             src_v, dst_v, rows0_v, rows1_v, sem, shared):
    cid = lax.axis_index("c")
    sid = lax.axis_index("s")
    wid = sid * NC + cid
    row0 = sid * ROWS_PER_TILE

    pltpu.sync_copy(zero_hbm.at[pl.ds(row0, ROWS_PER_TILE)],
                    shared.at[pl.ds(row0, ROWS_PER_TILE)])
    plsc.subcore_barrier()

    rows = (rows0_v, rows1_v)

    def gath(e, b):
        pltpu.async_copy(m_hbm.at[src_v.at[e]], rows[b], sem)

    def wait_g(e, b):
        pltpu.make_async_copy(m_hbm.at[src_v.at[e]], rows[b], sem).wait()

    def scat_sync(e, b):
        pltpu.sync_copy(rows[b], shared.at[dst_v.at[e]], add=True)

    for h in range(CPT // HALF):
        pltpu.sync_copy(src_hbm.at[wid, pl.ds(h * HALF, HALF)], src_v)
        pltpu.sync_copy(dst_hbm.at[wid, pl.ds(h * HALF, HALF)], dst_v)

        gath(0, 0)
        gath(1, 1)

        def pair(i, carry):
            for b in range(2):
                e = 2 * i + b
                wait_g(e, b)
                scat_sync(e, b)
                gath(e + 2, b)
            return carry

        lax.fori_loop(0, (HALF - 2) // 2, pair, 0, unroll=False)

        wait_g(HALF - 2, 0)
        scat_sync(HALF - 2, 0)
        wait_g(HALF - 1, 1)
        scat_sync(HALF - 1, 1)
    plsc.subcore_barrier()
    pltpu.sync_copy(shared.at[pl.ds(row0, ROWS_PER_TILE)],
                    out_hbm.at[cid, pl.ds(row0, ROWS_PER_TILE)])


_sc_scatter = pl.kernel(
    _sc_body,
    out_type=jax.ShapeDtypeStruct((NC, N_PAD, D), jnp.float32),
    mesh=plsc.VectorSubcoreMesh(core_axis_name="c", subcore_axis_name="s"),
    scratch_types=[
        pltpu.VMEM((HALF, CHUNK), jnp.int32),
        pltpu.VMEM((HALF, CHUNK), jnp.int32),
        pltpu.VMEM((CHUNK, D), jnp.float32),
        pltpu.VMEM((CHUNK, D), jnp.float32),
        pltpu.SemaphoreType.DMA,
        pltpu.VMEM_SHARED((N_PAD, D), jnp.float32),
    ],
)


def kernel(H, src, dst, W, gamma, beta):
    H2 = H.reshape(N_NODES, D)

    m = pl.pallas_call(
        _mm_body,
        out_shape=jax.ShapeDtypeStruct((N_NODES, D), jnp.float32),
        grid=(N_NODES // ROW_BLK,),
        in_specs=[pl.BlockSpec((ROW_BLK, D), lambda i: (i, 0)),
                  pl.BlockSpec((D, D), lambda i: (0, 0))],
        out_specs=pl.BlockSpec((ROW_BLK, D), lambda i: (i, 0)),
    )(H2, W)

    pad = EDGES_PAD - src.shape[0]
    pad_iota = lax.iota(jnp.int32, pad)
    src3 = jnp.concatenate(
        [src.astype(jnp.int32), pad_iota % N_NODES]
    ).reshape(NW, CPT, CHUNK)
    dst3 = jnp.concatenate(
        [dst.astype(jnp.int32), N_NODES + pad_iota % (N_PAD - N_NODES)]
    ).reshape(NW, CPT, CHUNK)
    zeros = jnp.zeros((N_PAD, D), jnp.float32)

    parts = _sc_scatter(m, src3, dst3, zeros)

    out = pl.pallas_call(
        _fin_body,
        out_shape=jax.ShapeDtypeStruct((N_NODES, D), jnp.float32),
        grid=(N_NODES // ROW_BLK,),
        in_specs=[pl.BlockSpec((ROW_BLK, D), lambda i: (i, 0)),
                  pl.BlockSpec((1, ROW_BLK, D), lambda i: (0, i, 0)),
                  pl.BlockSpec((1, ROW_BLK, D), lambda i: (1, i, 0)),
                  pl.BlockSpec((1, D), lambda i: (0, 0)),
                  pl.BlockSpec((1, D), lambda i: (0, 0))],
        out_specs=pl.BlockSpec((ROW_BLK, D), lambda i: (i, 0)),
    )(H2, parts, parts, gamma.reshape(1, D), beta.reshape(1, D))

    return out.reshape(1, N_NODES, D)


# speedup vs baseline: 3.8164x; 1.0048x over previous
import functools

import jax
import jax.numpy as jnp
from jax import lax
from jax.experimental import pallas as pl
from jax.experimental.pallas import tpu as pltpu
from jax.experimental.pallas import tpu_sc as plsc

D = 128
N_NODES = 10000
NC, NS = 2, 16
NW = NC * NS
ROWS_PER_TILE = 632
N_PAD = NS * ROWS_PER_TILE
CHUNK = 125
N_EDGES = 320000
CPT = 80
HALF = 40

ROW_BLK = 2000


def _mm_body(h_ref, w_ref, o_ref):
    o_ref[...] = lax.dot_general(
        h_ref[...], w_ref[...], (((1,), (1,)), ((), ())),
        preferred_element_type=jnp.float32)


def _fin_body(h_ref, a0_ref, a1_ref, g_ref, b_ref, o_ref):
    agg = a0_ref[0] + a1_ref[0]
    ge = 0.5 * agg * (1.0 + lax.erf(agg * 0.7071067811865476))
    x = h_ref[...] + ge
    mu = jnp.mean(x, axis=1, keepdims=True)
    xc = x - mu
    var = jnp.mean(xc * xc, axis=1, keepdims=True)
    y = xc * lax.rsqrt(var + 1e-5)
    o_ref[...] = y * g_ref[...] + b_ref[...]


def _sc_body(m_hbm, src_hbm, dst_hbm, zero_hbm, out_hbm,
             src_v, dst_v, rows0_v, rows1_v, sem, shared):
    cid = lax.axis_index("c")
    sid = lax.axis_index("s")
    wid = sid * NC + cid
    row0 = sid * ROWS_PER_TILE

    pltpu.sync_copy(zero_hbm.at[pl.ds(row0, ROWS_PER_TILE)],
                    shared.at[pl.ds(row0, ROWS_PER_TILE)])
    plsc.subcore_barrier()

    rows = (rows0_v, rows1_v)

    def gath(e, b):
        pltpu.async_copy(m_hbm.at[src_v.at[e]], rows[b], sem)

    def wait_g(e, b):
        pltpu.make_async_copy(m_hbm.at[src_v.at[e]], rows[b], sem).wait()

    def scat_sync(e, b):
        pltpu.sync_copy(rows[b], shared.at[dst_v.at[e]], add=True)

    for h in range(CPT // HALF):
        pltpu.sync_copy(src_hbm.at[wid, pl.ds(h * HALF, HALF)], src_v)
        pltpu.sync_copy(dst_hbm.at[wid, pl.ds(h * HALF, HALF)], dst_v)

        gath(0, 0)
        gath(1, 1)

        def pair(i, carry):
            for b in range(2):
                e = 2 * i + b
                wait_g(e, b)
                scat_sync(e, b)
                gath(e + 2, b)
            return carry

        lax.fori_loop(0, (HALF - 2) // 2, pair, 0, unroll=False)

        wait_g(HALF - 2, 0)
        scat_sync(HALF - 2, 0)
        wait_g(HALF - 1, 1)
        scat_sync(HALF - 1, 1)
    plsc.subcore_barrier()
    pltpu.sync_copy(shared.at[pl.ds(row0, ROWS_PER_TILE)],
                    out_hbm.at[cid, pl.ds(row0, ROWS_PER_TILE)])


_sc_scatter = pl.kernel(
    _sc_body,
    out_type=jax.ShapeDtypeStruct((NC, N_PAD, D), jnp.float32),
    mesh=plsc.VectorSubcoreMesh(core_axis_name="c", subcore_axis_name="s"),
    scratch_types=[
        pltpu.VMEM((HALF, CHUNK), jnp.int32),
        pltpu.VMEM((HALF, CHUNK), jnp.int32),
        pltpu.VMEM((CHUNK, D), jnp.float32),
        pltpu.VMEM((CHUNK, D), jnp.float32),
        pltpu.SemaphoreType.DMA,
        pltpu.VMEM_SHARED((N_PAD, D), jnp.float32),
    ],
)


def kernel(H, src, dst, W, gamma, beta):
    H2 = H.reshape(N_NODES, D)

    m = pl.pallas_call(
        _mm_body,
        out_shape=jax.ShapeDtypeStruct((N_NODES, D), jnp.float32),
        grid=(N_NODES // ROW_BLK,),
        in_specs=[pl.BlockSpec((ROW_BLK, D), lambda i: (i, 0)),
                  pl.BlockSpec((D, D), lambda i: (0, 0))],
        out_specs=pl.BlockSpec((ROW_BLK, D), lambda i: (i, 0)),
    )(H2, W)

    src3 = src.astype(jnp.int32).reshape(NW, CPT, CHUNK)
    dst3 = dst.astype(jnp.int32).reshape(NW, CPT, CHUNK)
    zeros = jnp.zeros((N_PAD, D), jnp.float32)

    parts = _sc_scatter(m, src3, dst3, zeros)

    out = pl.pallas_call(
        _fin_body,
        out_shape=jax.ShapeDtypeStruct((N_NODES, D), jnp.float32),
        grid=(N_NODES // ROW_BLK,),
        in_specs=[pl.BlockSpec((ROW_BLK, D), lambda i: (i, 0)),
                  pl.BlockSpec((1, ROW_BLK, D), lambda i: (0, i, 0)),
                  pl.BlockSpec((1, ROW_BLK, D), lambda i: (1, i, 0)),
                  pl.BlockSpec((1, D), lambda i: (0, 0)),
                  pl.BlockSpec((1, D), lambda i: (0, 0))],
        out_specs=pl.BlockSpec((ROW_BLK, D), lambda i: (i, 0)),
    )(H2, parts, parts, gamma.reshape(1, D), beta.reshape(1, D))

    return out.reshape(1, N_NODES, D)
